# baseline (speedup 1.0000x reference)
import functools

import jax
import jax.numpy as jnp
from jax.experimental import pallas as pl
from jax.experimental.pallas import tpu as pltpu

K = 5
BI = 200
NC = 1280


def _pass1(*refs):
    x_refs = refs[0:K]
    h_refs = refs[K:2 * K]
    w_ref, b_ref, zT_ref, acc_ref = refs[2 * K:]
    i = pl.program_id(0)
    n_steps = pl.num_programs(0)
    M = acc_ref.shape[1]

    @pl.when(i == 0)
    def _():
        acc_ref[...] = jnp.zeros(acc_ref.shape, jnp.float32)

    for c in range(K):
        xt = jnp.dot(x_refs[c][...], w_ref[...],
                     preferred_element_type=jnp.float32) + b_ref[...]
        dv = jnp.sum(h_refs[c][...], axis=1, keepdims=True)
        dvs = jnp.where(dv > 0, jax.lax.rsqrt(dv), 0.0)
        xa = jnp.concatenate([xt * dvs, jnp.ones((xt.shape[0], 1),
                                                 jnp.float32)], axis=1)
        for n0 in range(0, M, NC):
            nc = min(NC, M - n0)
            p = jax.lax.dot_general(xa, h_refs[c][:, n0:n0 + nc],
                                    (((0,), (0,)), ((), ())),
                                    preferred_element_type=jnp.float32)
            acc_ref[:, n0:n0 + nc] += p

    @pl.when(i == n_steps - 1)
    def _():
        zT_ref[...] = acc_ref[...]


def _pass2(*refs):
    h_refs = refs[0:K]
    zT_ref, g_ref, be_ref, o_ref, zs_ref = refs[K:]
    i = pl.program_id(0)
    dout = zs_ref.shape[0]

    @pl.when(i == 0)
    def _():
        de = zT_ref[dout:dout + 1, :]
        dei = jnp.where(de > 0, 1.0 / de, 0.0)
        zs_ref[...] = zT_ref[0:dout, :] * dei

    for c in range(K):
        h = h_refs[c][...]
        y = jax.lax.dot_general(h, zs_ref[...], (((1,), (1,)), ((), ())),
                                preferred_element_type=jnp.float32)
        dv = jnp.sum(h, axis=1, keepdims=True)
        dvs = jnp.where(dv > 0, jax.lax.rsqrt(dv), 0.0)
        y = y * dvs
        mean = jnp.mean(y, axis=1, keepdims=True)
        cen = y - mean
        var = jnp.mean(cen * cen, axis=1, keepdims=True)
        yn = cen * jax.lax.rsqrt(var + 1e-5) * g_ref[...] + be_ref[...]
        o_ref[pl.ds(c * BI, BI), :] = jnp.maximum(yn, 0.0)


def _row_spec(shape_cols, c):
    return pl.BlockSpec((BI, shape_cols), lambda i, c=c: (K * i + c, 0))


@functools.partial(jax.jit, static_argnames=())
def kernel(x, H, W, b, gamma, beta):
    N, DIN = x.shape
    M = H.shape[1]
    DOUT = W.shape[1]
    grid = (N // (K * BI),)

    b2 = b.reshape(1, DOUT)
    g2 = gamma.reshape(1, DOUT)
    be2 = beta.reshape(1, DOUT)

    zT = pl.pallas_call(
        _pass1,
        grid=grid,
        in_specs=(
            [_row_spec(DIN, c) for c in range(K)]
            + [_row_spec(M, c) for c in range(K)]
            + [pl.BlockSpec((DIN, DOUT), lambda i: (0, 0)),
               pl.BlockSpec((1, DOUT), lambda i: (0, 0))]
        ),
        out_specs=pl.BlockSpec((DOUT + 1, M), lambda i: (0, 0)),
        out_shape=jax.ShapeDtypeStruct((DOUT + 1, M), jnp.float32),
        scratch_shapes=[pltpu.VMEM((DOUT + 1, M), jnp.float32)],
    )(*([x] * K), *([H] * K), W, b2)

    outs = pl.pallas_call(
        _pass2,
        grid=grid,
        in_specs=(
            [_row_spec(M, c) for c in range(K)]
            + [pl.BlockSpec((DOUT + 1, M), lambda i: (0, 0)),
               pl.BlockSpec((1, DOUT), lambda i: (0, 0)),
               pl.BlockSpec((1, DOUT), lambda i: (0, 0))]
        ),
        out_specs=pl.BlockSpec((K * BI, DOUT), lambda i: (i, 0)),
        out_shape=jax.ShapeDtypeStruct((N, DOUT), jnp.float32),
        scratch_shapes=[pltpu.VMEM((DOUT, M), jnp.float32)],
    )(*([H] * K), zT, g2, be2)

    return outs


# speedup vs baseline: 1.0072x; 1.0072x over previous
import functools

import jax
import jax.numpy as jnp
from jax.experimental import pallas as pl
from jax.experimental.pallas import tpu as pltpu

K = 5
BI = 200
B1 = 1000
NC = 1280


def _pass1(x_ref, h_ref, w_ref, b_ref, zT_ref, acc_ref):
    i = pl.program_id(0)
    n_steps = pl.num_programs(0)
    M = acc_ref.shape[1]

    xt = jnp.dot(x_ref[...], w_ref[...],
                 preferred_element_type=jnp.float32) + b_ref[...]
    dv = jnp.sum(h_ref[...], axis=1, keepdims=True)
    dvs = jnp.where(dv > 0, jax.lax.rsqrt(dv), 0.0)
    xa = jnp.concatenate([xt * dvs, jnp.ones((xt.shape[0], 1),
                                             jnp.float32)], axis=1)
    for n0 in range(0, M, NC):
        nc = min(NC, M - n0)
        p = jax.lax.dot_general(xa, h_ref[:, n0:n0 + nc],
                                (((0,), (0,)), ((), ())),
                                preferred_element_type=jnp.float32)

        @pl.when(i == 0)
        def _():
            acc_ref[:, n0:n0 + nc] = p

        @pl.when(i > 0)
        def _():
            acc_ref[:, n0:n0 + nc] += p

    @pl.when(i == n_steps - 1)
    def _():
        zT_ref[...] = acc_ref[...]


def _pass2(*refs):
    h_refs = refs[0:K]
    zT_ref, g_ref, be_ref, o_ref, zs_ref = refs[K:]
    i = pl.program_id(0)
    dout = zs_ref.shape[0]

    @pl.when(i == 0)
    def _():
        de = zT_ref[dout:dout + 1, :]
        dei = jnp.where(de > 0, 1.0 / de, 0.0)
        zs_ref[...] = zT_ref[0:dout, :] * dei

    for c in range(K):
        h = h_refs[c][...]
        y = jax.lax.dot_general(h, zs_ref[...], (((1,), (1,)), ((), ())),
                                preferred_element_type=jnp.float32)
        dv = jnp.sum(h, axis=1, keepdims=True)
        dvs = jnp.where(dv > 0, jax.lax.rsqrt(dv), 0.0)
        y = y * dvs
        mean = jnp.mean(y, axis=1, keepdims=True)
        cen = y - mean
        var = jnp.mean(cen * cen, axis=1, keepdims=True)
        yn = cen * jax.lax.rsqrt(var + 1e-5) * g_ref[...] + be_ref[...]
        o_ref[pl.ds(c * BI, BI), :] = jnp.maximum(yn, 0.0)


def _row_spec(shape_cols, c):
    return pl.BlockSpec((BI, shape_cols), lambda i, c=c: (K * i + c, 0))


@functools.partial(jax.jit, static_argnames=())
def kernel(x, H, W, b, gamma, beta):
    N, DIN = x.shape
    M = H.shape[1]
    DOUT = W.shape[1]
    grid = (N // (K * BI),)

    b2 = b.reshape(1, DOUT)
    g2 = gamma.reshape(1, DOUT)
    be2 = beta.reshape(1, DOUT)

    zT = pl.pallas_call(
        _pass1,
        grid=(N // B1,),
        in_specs=[
            pl.BlockSpec((B1, DIN), lambda i: (i, 0)),
            pl.BlockSpec((B1, M), lambda i: (i, 0)),
            pl.BlockSpec((DIN, DOUT), lambda i: (0, 0)),
            pl.BlockSpec((1, DOUT), lambda i: (0, 0)),
        ],
        out_specs=pl.BlockSpec((DOUT + 1, M), lambda i: (0, 0)),
        out_shape=jax.ShapeDtypeStruct((DOUT + 1, M), jnp.float32),
        scratch_shapes=[pltpu.VMEM((DOUT + 1, M), jnp.float32)],
    )(x, H, W, b2)

    outs = pl.pallas_call(
        _pass2,
        grid=grid,
        in_specs=(
            [_row_spec(M, c) for c in range(K)]
            + [pl.BlockSpec((DOUT + 1, M), lambda i: (0, 0)),
               pl.BlockSpec((1, DOUT), lambda i: (0, 0)),
               pl.BlockSpec((1, DOUT), lambda i: (0, 0))]
        ),
        out_specs=pl.BlockSpec((K * BI, DOUT), lambda i: (i, 0)),
        out_shape=jax.ShapeDtypeStruct((N, DOUT), jnp.float32),
        scratch_shapes=[pltpu.VMEM((DOUT, M), jnp.float32)],
    )(*([H] * K), zT, g2, be2)

    return outs


# speedup vs baseline: 1.0247x; 1.0174x over previous
import functools

import jax
import jax.numpy as jnp
from jax.experimental import pallas as pl
from jax.experimental.pallas import tpu as pltpu

B = 1000
NC = 1280


def _fused(x_ref, h_ref, w_ref, b_ref, g_ref, be_ref, o_ref, acc_ref, zs_ref):
    ph = pl.program_id(0)
    i = pl.program_id(1)
    dout = zs_ref.shape[0]
    M = acc_ref.shape[1]

    @pl.when(ph == 0)
    def _():
        xt = jnp.dot(x_ref[...], w_ref[...],
                     preferred_element_type=jnp.float32) + b_ref[...]
        dv = jnp.sum(h_ref[...], axis=1, keepdims=True)
        dvs = jnp.where(dv > 0, jax.lax.rsqrt(dv), 0.0)
        xa = jnp.concatenate([xt * dvs, jnp.ones((xt.shape[0], 1),
                                                 jnp.float32)], axis=1)
        for n0 in range(0, M, NC):
            nc = min(NC, M - n0)
            p = jax.lax.dot_general(xa, h_ref[:, n0:n0 + nc],
                                    (((0,), (0,)), ((), ())),
                                    preferred_element_type=jnp.float32)

            @pl.when(i == 0)
            def _():
                acc_ref[:, n0:n0 + nc] = p

            @pl.when(i > 0)
            def _():
                acc_ref[:, n0:n0 + nc] += p

    @pl.when(ph == 1)
    def _():
        @pl.when(i == 0)
        def _():
            de = acc_ref[dout:dout + 1, :]
            dei = jnp.where(de > 0, 1.0 / de, 0.0)
            zs_ref[...] = acc_ref[0:dout, :] * dei

        h = h_ref[...]
        y = jax.lax.dot_general(h, zs_ref[...], (((1,), (1,)), ((), ())),
                                preferred_element_type=jnp.float32)
        dv = jnp.sum(h, axis=1, keepdims=True)
        dvs = jnp.where(dv > 0, jax.lax.rsqrt(dv), 0.0)
        y = y * dvs
        mean = jnp.mean(y, axis=1, keepdims=True)
        cen = y - mean
        var = jnp.mean(cen * cen, axis=1, keepdims=True)
        yn = cen * jax.lax.rsqrt(var + 1e-5) * g_ref[...] + be_ref[...]
        o_ref[...] = jnp.maximum(yn, 0.0)


@functools.partial(jax.jit, static_argnames=())
def kernel(x, H, W, b, gamma, beta):
    N, DIN = x.shape
    M = H.shape[1]
    DOUT = W.shape[1]

    b2 = b.reshape(1, DOUT)
    g2 = gamma.reshape(1, DOUT)
    be2 = beta.reshape(1, DOUT)

    out = pl.pallas_call(
        _fused,
        grid=(2, N // B),
        in_specs=[
            pl.BlockSpec((B, DIN), lambda p, i: (i, 0)),
            pl.BlockSpec((B, M), lambda p, i: (i, 0)),
            pl.BlockSpec((DIN, DOUT), lambda p, i: (0, 0)),
            pl.BlockSpec((1, DOUT), lambda p, i: (0, 0)),
            pl.BlockSpec((1, DOUT), lambda p, i: (0, 0)),
            pl.BlockSpec((1, DOUT), lambda p, i: (0, 0)),
        ],
        out_specs=pl.BlockSpec((B, DOUT), lambda p, i: (i, 0)),
        out_shape=jax.ShapeDtypeStruct((N, DOUT), jnp.float32),
        scratch_shapes=[pltpu.VMEM((DOUT + 1, M), jnp.float32),
                        pltpu.VMEM((DOUT, M), jnp.float32)],
    )(x, H, W, b2, g2, be2)

    return out
